# Initial kernel scaffold; baseline (speedup 1.0000x reference)
import jax
import jax.numpy as jnp
from jax.experimental import pallas as pl


def kernel(x, scale, Wq, bq, Wk, bk, Wv, bv):
    raise NotImplementedError("write your pallas kernel here")


# speedup vs baseline: 22.2683x; 22.2683x over previous
import functools

import jax
import jax.numpy as jnp
from jax import lax
from jax.experimental import pallas as pl

_K = 8


def _routing_body(x_ref, scale_ref, wq_ref, bq_ref, wk_ref, bk_ref,
                  wv_ref, bv_ref, o_ref, *, n, d, k):
    xb = x_ref[0]
    qT = jnp.dot(wq_ref[...], xb, preferred_element_type=jnp.float32) + bq_ref[...]
    kT = jnp.dot(wk_ref[...], xb, preferred_element_type=jnp.float32) + bk_ref[...]
    vT = jnp.dot(wv_ref[...], xb, preferred_element_type=jnp.float32) + bv_ref[...]

    sim = lax.dot_general(qT, kT, (((0,), (0,)), ((), ())),
                          preferred_element_type=jnp.float32)
    sim = sim * (1.0 / (d ** 0.5))
    row = lax.broadcasted_iota(jnp.int32, (n, n), 0)
    col = lax.broadcasted_iota(jnp.int32, (n, n), 1)
    sim = jnp.where(row == col, sim - 1e9, sim)

    work = sim
    sel = jnp.zeros((n, n), jnp.float32)
    m0 = None
    for it in range(k):
        m = jnp.max(work, axis=1, keepdims=True)
        if it == 0:
            m0 = m
        cand = jnp.where(work == m, col, n)
        amin = jnp.min(cand, axis=1, keepdims=True)
        onehot = col == amin
        sel = jnp.where(onehot, 1.0, sel)
        work = jnp.where(onehot, -jnp.inf, work)

    e = jnp.exp(sim - m0) * sel
    denom = jnp.sum(e, axis=1)
    comb = lax.dot_general(vT, e, (((1,), (1,)), ((), ())),
                           preferred_element_type=jnp.float32)
    comb = comb * (1.0 / denom)[None, :]
    o_ref[0] = xb + scale_ref[0, 0] * comb


def kernel(x, scale, Wq, bq, Wk, bk, Wv, bv):
    B_, C_, H_, W_ = x.shape
    N = H_ * W_
    D_ = Wq.shape[0]
    xr = x.reshape(B_, C_, N)
    body = functools.partial(_routing_body, n=N, d=D_, k=_K)
    out = pl.pallas_call(
        body,
        grid=(B_,),
        in_specs=[
            pl.BlockSpec((1, C_, N), lambda b: (b, 0, 0)),
            pl.BlockSpec((1, 1), lambda b: (0, 0)),
            pl.BlockSpec((D_, C_), lambda b: (0, 0)),
            pl.BlockSpec((D_, 1), lambda b: (0, 0)),
            pl.BlockSpec((D_, C_), lambda b: (0, 0)),
            pl.BlockSpec((D_, 1), lambda b: (0, 0)),
            pl.BlockSpec((C_, C_), lambda b: (0, 0)),
            pl.BlockSpec((C_, 1), lambda b: (0, 0)),
        ],
        out_specs=pl.BlockSpec((1, C_, N), lambda b: (b, 0, 0)),
        out_shape=jax.ShapeDtypeStruct((B_, C_, N), jnp.float32),
    )(xr, scale.reshape(1, 1), Wq, bq.reshape(D_, 1), Wk, bk.reshape(D_, 1),
      Wv, bv.reshape(C_, 1))
    return out.reshape(B_, C_, H_, W_)


# speedup vs baseline: 44.6724x; 2.0061x over previous
import functools

import jax
import jax.numpy as jnp
from jax import lax
from jax.experimental import pallas as pl

_K = 8


def _routing_body(x_ref, scale_ref, wq_ref, bq_ref, wk_ref, bk_ref,
                  wv_ref, bv_ref, o_ref, *, n, d, k):
    xb = x_ref[0]
    qT = jnp.dot(wq_ref[...], xb, preferred_element_type=jnp.float32) + bq_ref[...]
    kT = jnp.dot(wk_ref[...], xb, preferred_element_type=jnp.float32) + bk_ref[...]
    vT = jnp.dot(wv_ref[...], xb, preferred_element_type=jnp.float32) + bv_ref[...]

    sim = lax.dot_general(qT, kT, (((0,), (0,)), ((), ())),
                          preferred_element_type=jnp.float32)
    sim = sim * (1.0 / (d ** 0.5))
    row = lax.broadcasted_iota(jnp.int32, (n, n), 0)
    col = lax.broadcasted_iota(jnp.int32, (n, n), 1)
    sim = jnp.where(row == col, sim - 1e9, sim)

    work = sim
    m0 = None
    for it in range(k):
        m = jnp.max(work, axis=1, keepdims=True)
        if it == 0:
            m0 = m
        if it < k - 1:
            work = jnp.where(work == m, -jnp.inf, work)
        else:
            selected = (work == -jnp.inf) | (work == m)

    e = jnp.where(selected, jnp.exp(sim - m0), 0.0)
    denom = jnp.sum(e, axis=1)
    comb = lax.dot_general(vT, e, (((1,), (1,)), ((), ())),
                           preferred_element_type=jnp.float32)
    comb = comb * (1.0 / denom)[None, :]
    o_ref[0] = xb + scale_ref[0, 0] * comb


def kernel(x, scale, Wq, bq, Wk, bk, Wv, bv):
    B_, C_, H_, W_ = x.shape
    N = H_ * W_
    D_ = Wq.shape[0]
    xr = x.reshape(B_, C_, N)
    body = functools.partial(_routing_body, n=N, d=D_, k=_K)
    out = pl.pallas_call(
        body,
        grid=(B_,),
        in_specs=[
            pl.BlockSpec((1, C_, N), lambda b: (b, 0, 0)),
            pl.BlockSpec((1, 1), lambda b: (0, 0)),
            pl.BlockSpec((D_, C_), lambda b: (0, 0)),
            pl.BlockSpec((D_, 1), lambda b: (0, 0)),
            pl.BlockSpec((D_, C_), lambda b: (0, 0)),
            pl.BlockSpec((D_, 1), lambda b: (0, 0)),
            pl.BlockSpec((C_, C_), lambda b: (0, 0)),
            pl.BlockSpec((C_, 1), lambda b: (0, 0)),
        ],
        out_specs=pl.BlockSpec((1, C_, N), lambda b: (b, 0, 0)),
        out_shape=jax.ShapeDtypeStruct((B_, C_, N), jnp.float32),
    )(xr, scale.reshape(1, 1), Wq, bq.reshape(D_, 1), Wk, bk.reshape(D_, 1),
      Wv, bv.reshape(C_, 1))
    return out.reshape(B_, C_, H_, W_)
